# baseline (speedup 1.0000x reference)
import functools

import jax
import jax.numpy as jnp
from jax import lax
from jax.experimental import pallas as pl
from jax.experimental.pallas import tpu as pltpu
from jax.experimental.pallas import tpu_sc as plsc

N_NODES = 10000
N_EDGES = 320000
D = 128
N_CLASSES = 40
N_LAYERS = 3
BN_EPS = 1e-5

NC = 2
NS = 16
BATCH = 128
NB_TOT = 160
NB_F = 80
NB_S = NB_TOT - NB_F
CH = 16
NBUF = 2
N_PAD = 10112
ROWS_PER_TILE = N_PAD // NS
TRASH_ROW = N_NODES


def _sc_agg_body(h_hbm, srcs_hbm, dsts_hbm, zeros_hbm, out_hbm,
                 idx_s_v, idx_d_v, rows_v, agg_sh, sems, isems):
    c = lax.axis_index("c")
    s = lax.axis_index("s")

    pltpu.sync_copy(zeros_hbm.at[pl.ds(s * ROWS_PER_TILE, ROWS_PER_TILE)],
                    agg_sh.at[pl.ds(s * ROWS_PER_TILE, ROWS_PER_TILE)])
    plsc.subcore_barrier()

    def stage(start, k, slot, wait):
        if wait:
            pltpu.make_async_copy(srcs_hbm.at[s, pl.ds(0, CH)],
                                  idx_s_v.at[slot], isems.at[0]).wait()
            pltpu.make_async_copy(dsts_hbm.at[s, pl.ds(0, CH)],
                                  idx_d_v.at[slot], isems.at[1]).wait()
        else:
            pltpu.async_copy(srcs_hbm.at[s, pl.ds(start + k * CH, CH)],
                             idx_s_v.at[slot], isems.at[0])
            pltpu.async_copy(dsts_hbm.at[s, pl.ds(start + k * CH, CH)],
                             idx_d_v.at[slot], isems.at[1])

    def run_range(start, nbatch):
        nch = nbatch // CH
        stage(start, 0, 0, False)
        stage(start, 0, 0, True)
        if nch > 1:
            stage(start, 1, 1, False)
        pltpu.async_copy(h_hbm.at[idx_s_v.at[0, 0]], rows_v.at[0], sems.at[0])

        def body(j, _):
            p = lax.rem(j, NBUF)
            ji = j + NBUF - 1
            cn = ji // CH

            @pl.when((ji < nbatch) & (lax.rem(ji, CH) == 0))
            def _():
                stage(start, cn, lax.rem(cn, 3), True)

                @pl.when(cn + 1 < nch)
                def _():
                    stage(start, cn + 1, lax.rem(cn + 1, 3), False)

            @pl.when(ji < nbatch)
            def _():
                pltpu.async_copy(
                    h_hbm.at[idx_s_v.at[lax.rem(cn, 3), lax.rem(ji, CH)]],
                    rows_v.at[lax.rem(ji, NBUF)], sems.at[lax.rem(ji, NBUF)])

            pltpu.make_async_copy(h_hbm.at[idx_s_v.at[0, 0]],
                                  rows_v.at[p], sems.at[p]).wait()
            pltpu.sync_copy(
                rows_v.at[p],
                agg_sh.at[idx_d_v.at[lax.rem(j // CH, 3), lax.rem(j, CH)]],
                add=True)
            return 0

        lax.fori_loop(0, nbatch, body, 0)

    if NB_F > 0:
        @pl.when(c == 0)
        def _():
            run_range(0, NB_F)

    if NB_S > 0:
        @pl.when(c == 1)
        def _():
            run_range(NB_F, NB_S)

    plsc.subcore_barrier()

    pltpu.sync_copy(agg_sh.at[pl.ds(s * ROWS_PER_TILE, ROWS_PER_TILE)],
                    out_hbm.at[c, pl.ds(s * ROWS_PER_TILE, ROWS_PER_TILE)])


@jax.jit
def _sc_agg(h, srcs, dsts, zeros):
    mesh = plsc.VectorSubcoreMesh(core_axis_name="c", subcore_axis_name="s")
    return pl.kernel(
        _sc_agg_body,
        mesh=mesh,
        out_type=jax.ShapeDtypeStruct((NC, N_PAD, D), jnp.float32),
        scratch_types=[
            pltpu.VMEM((3, CH, BATCH), jnp.int32),
            pltpu.VMEM((3, CH, BATCH), jnp.int32),
            pltpu.VMEM((NBUF, BATCH, D), jnp.float32),
            pltpu.VMEM_SHARED((N_PAD, D), jnp.float32),
            pltpu.SemaphoreType.DMA((NBUF,)),
            pltpu.SemaphoreType.DMA((2,)),
        ],
    )(h, srcs, dsts, zeros)


def _mlp_bn(x, w1, b1, w2, b2, g, bt):
    y = jnp.dot(x, w1, preferred_element_type=jnp.float32) + b1
    y = jnp.maximum(y, 0.0)
    y = jnp.dot(y, w2, preferred_element_type=jnp.float32) + b2
    y = jnp.maximum(y, 0.0)
    mean = jnp.mean(y, axis=0, keepdims=True)
    var = jnp.mean(jnp.square(y - mean), axis=0, keepdims=True)
    return (y - mean) * (g * lax.rsqrt(var + BN_EPS)) + bt


def _tc_mid_body(h_ref, p_ref, w1_ref, b1_ref, w2_ref, b2_ref, g_ref, bt_ref,
                 o_ref):
    x = h_ref[...] + p_ref[0, :N_NODES, :] + p_ref[1, :N_NODES, :]
    o_ref[...] = _mlp_bn(x, w1_ref[...], b1_ref[...], w2_ref[...], b2_ref[...],
                         g_ref[...], bt_ref[...])


def _tc_last_body(h_ref, p_ref, w1_ref, b1_ref, w2_ref, b2_ref, g_ref, bt_ref,
                  fc1w_ref, fc1b_ref, fc2w_ref, fc2b_ref, o_ref):
    x = h_ref[...] + p_ref[0, :N_NODES, :] + p_ref[1, :N_NODES, :]
    hh = _mlp_bn(x, w1_ref[...], b1_ref[...], w2_ref[...], b2_ref[...],
                 g_ref[...], bt_ref[...])
    z = jnp.dot(hh, fc1w_ref[...], preferred_element_type=jnp.float32)
    z = jnp.maximum(z + fc1b_ref[...], 0.0)
    logits = jnp.dot(z, fc2w_ref[...],
                     preferred_element_type=jnp.float32) + fc2b_ref[...]
    col = lax.broadcasted_iota(jnp.int32, logits.shape, 1)
    zm = jnp.where(col < N_CLASSES, logits, -jnp.inf)
    m = jnp.max(zm, axis=-1, keepdims=True)
    lse = m + jnp.log(jnp.sum(jnp.exp(zm - m), axis=-1, keepdims=True))
    o_ref[...] = (logits - lse)[:, :N_CLASSES]


def _tc_mid(h, p, w1, b1, w2, b2, g, bt):
    return pl.pallas_call(
        _tc_mid_body,
        out_shape=jax.ShapeDtypeStruct((N_NODES, D), jnp.float32),
    )(h, p, w1, b1, w2, b2, g, bt)


def _tc_last(h, p, w1, b1, w2, b2, g, bt, fc1w, fc1b, fc2w, fc2b):
    return pl.pallas_call(
        _tc_last_body,
        out_shape=jax.ShapeDtypeStruct((N_NODES, N_CLASSES), jnp.float32),
    )(h, p, w1, b1, w2, b2, g, bt, fc1w, fc1b, fc2w, fc2b)


def kernel(features, edge_index, W1, b1, W2, b2, bn_gamma, bn_beta,
           fc1_W, fc1_b, fc2_W, fc2_b):
    src = edge_index[0].astype(jnp.int32)
    dst = edge_index[1].astype(jnp.int32)
    pad = NS * NB_TOT * BATCH - N_EDGES
    pad_iota = jnp.arange(pad, dtype=jnp.int32)
    srcs = jnp.concatenate([src, pad_iota & 8191])
    dsts = jnp.concatenate([dst, TRASH_ROW + (pad_iota & 63)])
    srcs = srcs.reshape(NS, NB_TOT, BATCH)
    dsts = dsts.reshape(NS, NB_TOT, BATCH)
    zeros = jnp.zeros((N_PAD, D), jnp.float32)

    fc2w_p = jnp.zeros((D, D), jnp.float32).at[:, :N_CLASSES].set(fc2_W)
    fc2b_p = jnp.zeros((D,), jnp.float32).at[:N_CLASSES].set(fc2_b)

    h = features
    for i in range(N_LAYERS - 1):
        p = _sc_agg(h, srcs, dsts, zeros)
        h = _tc_mid(h, p, W1[i], b1[i], W2[i], b2[i], bn_gamma[i], bn_beta[i])
    i = N_LAYERS - 1
    p = _sc_agg(h, srcs, dsts, zeros)
    return _tc_last(h, p, W1[i], b1[i], W2[i], b2[i], bn_gamma[i], bn_beta[i],
                    fc1_W, fc1_b, fc2w_p, fc2b_p)


# speedup vs baseline: 1.0303x; 1.0303x over previous
import functools

import jax
import jax.numpy as jnp
from jax import lax
from jax.experimental import pallas as pl
from jax.experimental.pallas import tpu as pltpu
from jax.experimental.pallas import tpu_sc as plsc

N_NODES = 10000
N_EDGES = 320000
D = 128
N_CLASSES = 40
N_LAYERS = 3
BN_EPS = 1e-5

NC = 2
NS = 16
BATCH = 128
NB_TOT = 160
NB_F = 80
NB_S = NB_TOT - NB_F
CH = 16
NBUF = 2
N_PAD = 10112
ROWS_PER_TILE = N_PAD // NS
TRASH_ROW = N_NODES


def _sc_agg_body(h_hbm, srcs_hbm, dsts_hbm, zeros_hbm, out_hbm,
                 idx_s_v, idx_d_v, rows_v, agg_sh, sems, isems):
    c = lax.axis_index("c")
    s = lax.axis_index("s")

    pltpu.sync_copy(zeros_hbm.at[pl.ds(s * ROWS_PER_TILE, ROWS_PER_TILE)],
                    agg_sh.at[pl.ds(s * ROWS_PER_TILE, ROWS_PER_TILE)])
    plsc.subcore_barrier()

    def stage(start, k, slot, wait):
        if wait:
            pltpu.make_async_copy(srcs_hbm.at[s, pl.ds(0, CH)],
                                  idx_s_v.at[slot], isems.at[0]).wait()
            pltpu.make_async_copy(dsts_hbm.at[s, pl.ds(0, CH)],
                                  idx_d_v.at[slot], isems.at[1]).wait()
        else:
            pltpu.async_copy(srcs_hbm.at[s, pl.ds(start + k * CH, CH)],
                             idx_s_v.at[slot], isems.at[0])
            pltpu.async_copy(dsts_hbm.at[s, pl.ds(start + k * CH, CH)],
                             idx_d_v.at[slot], isems.at[1])

    def run_range(start, nbatch):
        nch = nbatch // CH
        stage(start, 0, 0, False)
        stage(start, 0, 0, True)
        if nch > 1:
            stage(start, 1, 1, False)
        pltpu.async_copy(h_hbm.at[idx_s_v.at[0, 0]], rows_v.at[0], sems.at[0])

        def body(j, _):
            p = lax.rem(j, NBUF)
            ji = j + NBUF - 1
            cn = ji // CH

            @pl.when((ji < nbatch) & (lax.rem(ji, CH) == 0))
            def _():
                stage(start, cn, lax.rem(cn, 3), True)

                @pl.when(cn + 1 < nch)
                def _():
                    stage(start, cn + 1, lax.rem(cn + 1, 3), False)

            @pl.when(ji < nbatch)
            def _():
                pltpu.async_copy(
                    h_hbm.at[idx_s_v.at[lax.rem(cn, 3), lax.rem(ji, CH)]],
                    rows_v.at[lax.rem(ji, NBUF)], sems.at[lax.rem(ji, NBUF)])

            pltpu.make_async_copy(h_hbm.at[idx_s_v.at[0, 0]],
                                  rows_v.at[p], sems.at[p]).wait()
            pltpu.sync_copy(
                rows_v.at[p],
                agg_sh.at[idx_d_v.at[lax.rem(j // CH, 3), lax.rem(j, CH)]],
                add=True)
            return 0

        lax.fori_loop(0, nbatch, body, 0)

    if NB_F > 0:
        @pl.when(c == 0)
        def _():
            run_range(0, NB_F)

    if NB_S > 0:
        @pl.when(c == 1)
        def _():
            run_range(NB_F, NB_S)

    plsc.subcore_barrier()

    pltpu.sync_copy(agg_sh.at[pl.ds(s * ROWS_PER_TILE, ROWS_PER_TILE)],
                    out_hbm.at[c, pl.ds(s * ROWS_PER_TILE, ROWS_PER_TILE)])


@jax.jit
def _sc_agg(h, srcs, dsts, zeros):
    mesh = plsc.VectorSubcoreMesh(core_axis_name="c", subcore_axis_name="s")
    return pl.kernel(
        _sc_agg_body,
        mesh=mesh,
        out_type=jax.ShapeDtypeStruct((NC, N_PAD, D), jnp.float32),
        scratch_types=[
            pltpu.VMEM((3, CH, BATCH), jnp.int32),
            pltpu.VMEM((3, CH, BATCH), jnp.int32),
            pltpu.VMEM((NBUF, BATCH, D), jnp.float32),
            pltpu.VMEM_SHARED((N_PAD, D), jnp.float32),
            pltpu.SemaphoreType.DMA((NBUF,)),
            pltpu.SemaphoreType.DMA((2,)),
        ],
    )(h, srcs, dsts, zeros)


def _prep_body(e_ref, srcs_ref, dsts_ref):
    pad_rows = NS * NB_TOT * BATCH // 128 - N_EDGES // 128
    i0 = lax.broadcasted_iota(jnp.int32, (pad_rows, 128), 0)
    i1 = lax.broadcasted_iota(jnp.int32, (pad_rows, 128), 1)
    flat = i0 * 128 + i1
    src = jnp.concatenate([e_ref[0].reshape(N_EDGES // 128, 128),
                           flat & 8191], axis=0)
    dst = jnp.concatenate([e_ref[1].reshape(N_EDGES // 128, 128),
                           TRASH_ROW + (flat & 63)], axis=0)
    srcs_ref[...] = src.reshape(NS, NB_TOT, BATCH)
    dsts_ref[...] = dst.reshape(NS, NB_TOT, BATCH)


def _prep(edge_index):
    return pl.pallas_call(
        _prep_body,
        out_shape=(jax.ShapeDtypeStruct((NS, NB_TOT, BATCH), jnp.int32),
                   jax.ShapeDtypeStruct((NS, NB_TOT, BATCH), jnp.int32)),
    )(edge_index)


def _mlp_bn(x, w1, b1, w2, b2, g, bt):
    y = jnp.dot(x, w1, preferred_element_type=jnp.float32) + b1
    y = jnp.maximum(y, 0.0)
    y = jnp.dot(y, w2, preferred_element_type=jnp.float32) + b2
    y = jnp.maximum(y, 0.0)
    mean = jnp.mean(y, axis=0, keepdims=True)
    var = jnp.mean(jnp.square(y - mean), axis=0, keepdims=True)
    return (y - mean) * (g * lax.rsqrt(var + BN_EPS)) + bt


def _tc_mid_body(h_ref, p_ref, w1_ref, b1_ref, w2_ref, b2_ref, g_ref, bt_ref,
                 o_ref):
    x = h_ref[...] + p_ref[0, :N_NODES, :] + p_ref[1, :N_NODES, :]
    o_ref[...] = _mlp_bn(x, w1_ref[...], b1_ref[...], w2_ref[...], b2_ref[...],
                         g_ref[...], bt_ref[...])


def _tc_last_body(h_ref, p_ref, w1_ref, b1_ref, w2_ref, b2_ref, g_ref, bt_ref,
                  fc1w_ref, fc1b_ref, fc2w_ref, fc2b_ref, o_ref):
    x = h_ref[...] + p_ref[0, :N_NODES, :] + p_ref[1, :N_NODES, :]
    hh = _mlp_bn(x, w1_ref[...], b1_ref[...], w2_ref[...], b2_ref[...],
                 g_ref[...], bt_ref[...])
    z = jnp.dot(hh, fc1w_ref[...], preferred_element_type=jnp.float32)
    z = jnp.maximum(z + fc1b_ref[...], 0.0)
    logits = jnp.dot(z, fc2w_ref[...],
                     preferred_element_type=jnp.float32) + fc2b_ref[...]
    col = lax.broadcasted_iota(jnp.int32, logits.shape, 1)
    zm = jnp.where(col < N_CLASSES, logits, -jnp.inf)
    m = jnp.max(zm, axis=-1, keepdims=True)
    lse = m + jnp.log(jnp.sum(jnp.exp(zm - m), axis=-1, keepdims=True))
    o_ref[...] = (logits - lse)[:, :N_CLASSES]


def _tc_mid(h, p, w1, b1, w2, b2, g, bt):
    return pl.pallas_call(
        _tc_mid_body,
        out_shape=jax.ShapeDtypeStruct((N_NODES, D), jnp.float32),
    )(h, p, w1, b1, w2, b2, g, bt)


def _tc_last(h, p, w1, b1, w2, b2, g, bt, fc1w, fc1b, fc2w, fc2b):
    return pl.pallas_call(
        _tc_last_body,
        out_shape=jax.ShapeDtypeStruct((N_NODES, N_CLASSES), jnp.float32),
    )(h, p, w1, b1, w2, b2, g, bt, fc1w, fc1b, fc2w, fc2b)


def kernel(features, edge_index, W1, b1, W2, b2, bn_gamma, bn_beta,
           fc1_W, fc1_b, fc2_W, fc2_b):
    srcs, dsts = _prep(edge_index.astype(jnp.int32))
    zeros = jnp.zeros((N_PAD, D), jnp.float32)

    fc2w_p = jnp.zeros((D, D), jnp.float32).at[:, :N_CLASSES].set(fc2_W)
    fc2b_p = jnp.zeros((D,), jnp.float32).at[:N_CLASSES].set(fc2_b)

    h = features
    for i in range(N_LAYERS - 1):
        p = _sc_agg(h, srcs, dsts, zeros)
        h = _tc_mid(h, p, W1[i], b1[i], W2[i], b2[i], bn_gamma[i], bn_beta[i])
    i = N_LAYERS - 1
    p = _sc_agg(h, srcs, dsts, zeros)
    return _tc_last(h, p, W1[i], b1[i], W2[i], b2[i], bn_gamma[i], bn_beta[i],
                    fc1_W, fc1_b, fc2w_p, fc2b_p)
